# Initial kernel scaffold; baseline (speedup 1.0000x reference)
import jax
import jax.numpy as jnp
from jax.experimental import pallas as pl


def kernel(inputs, task_param, alpha, Wg, bg, Wt, bt, We, be):
    raise NotImplementedError("write your pallas kernel here")


# speedup vs baseline: 1.0367x; 1.0367x over previous
import functools

import jax
import jax.numpy as jnp
from jax.experimental import pallas as pl
from jax.experimental.pallas import tpu as pltpu

L, B, D, E = 8192, 2, 768, 8
N = L * B
TM = 512


def _moe_dense_body(alpha_ref, x_ref, wg_ref, bg_ref, tp_ref, wt_ref, bt_ref,
                    we_ref, be_ref, out_ref, laux_ref, psum_ref, cnt_ref):
    i = pl.program_id(0)
    nt = pl.num_programs(0)
    alpha = alpha_ref[0, 0]
    x = x_ref[...]

    task_logits = (jnp.dot(tp_ref[...], wt_ref[...],
                           preferred_element_type=jnp.float32)
                   + bt_ref[...])
    logits = ((1.0 - alpha)
              * (jnp.dot(x, wg_ref[...], preferred_element_type=jnp.float32)
                 + bg_ref[...])
              + alpha * task_logits)
    logits = jnp.where(jnp.isfinite(logits), logits, 0.0)

    mx = jnp.max(logits, axis=-1, keepdims=True)
    iota_e = jax.lax.broadcasted_iota(jnp.int32, (TM, E), 1)
    sel = jnp.min(jnp.where(logits == mx, iota_e, E), axis=-1,
                  keepdims=True)
    onehot = (iota_e == sel).astype(jnp.float32)

    ex = jnp.exp(logits - mx)
    probs = ex / jnp.sum(ex, axis=-1, keepdims=True)

    @pl.when(i == 0)
    def _init():
        psum_ref[...] = jnp.zeros_like(psum_ref)
        cnt_ref[...] = jnp.zeros_like(cnt_ref)

    psum_ref[...] += jnp.sum(probs, axis=0, keepdims=True)
    cnt_ref[...] += jnp.sum(onehot, axis=0, keepdims=True)

    acc = jnp.zeros((TM, D), jnp.float32)
    for e in range(E):
        w = onehot[:, e:e + 1]
        ye = (jnp.dot(x, we_ref[e], preferred_element_type=jnp.float32)
              + be_ref[e:e + 1, :])
        acc += w * ye
    out_ref[...] = acc

    @pl.when(i == nt - 1)
    def _fin():
        laux_ref[0, 0] = jnp.sum(psum_ref[...] * cnt_ref[...]) / (N * N)


@jax.jit
def kernel(inputs, task_param, alpha, Wg, bg, Wt, bt, We, be):
    x2 = inputs.reshape(N, D)
    alpha2 = jnp.asarray(alpha, jnp.float32).reshape(1, 1)
    out2, laux = pl.pallas_call(
        _moe_dense_body,
        grid=(N // TM,),
        in_specs=[
            pl.BlockSpec(memory_space=pltpu.SMEM),
            pl.BlockSpec((TM, D), lambda i: (i, 0)),
            pl.BlockSpec((D, E), lambda i: (0, 0)),
            pl.BlockSpec((1, E), lambda i: (0, 0)),
            pl.BlockSpec((1, D), lambda i: (0, 0)),
            pl.BlockSpec((D, E), lambda i: (0, 0)),
            pl.BlockSpec((1, E), lambda i: (0, 0)),
            pl.BlockSpec((E, D, D), lambda i: (0, 0, 0)),
            pl.BlockSpec((E, D), lambda i: (0, 0)),
        ],
        out_specs=[
            pl.BlockSpec((TM, D), lambda i: (i, 0)),
            pl.BlockSpec(memory_space=pltpu.SMEM),
        ],
        out_shape=[
            jax.ShapeDtypeStruct((N, D), jnp.float32),
            jax.ShapeDtypeStruct((1, 1), jnp.float32),
        ],
        scratch_shapes=[
            pltpu.VMEM((1, E), jnp.float32),
            pltpu.VMEM((1, E), jnp.float32),
        ],
    )(alpha2, x2, Wg, bg.reshape(1, E), task_param.reshape(1, D), Wt,
      bt.reshape(1, E), We, be)
    return out2.reshape(L, B, D), laux[0, 0]
